# baseline (speedup 1.0000x reference)
import functools

import jax
import jax.numpy as jnp
from jax import lax
from jax.experimental import pallas as pl
from jax.experimental.pallas import tpu as pltpu
from jax.experimental.pallas import tpu_sc as plsc
from jax._src.pallas import mpmd

_INFO = plsc.get_sparse_core_info()
_NC = _INFO.num_cores
_NS = _INFO.num_subcores
_NW = _NC * _NS

_SCS_ROWS = 1024

_S_CHUNK = 128
_S_NBUF = 3
_S_AHEAD = 2

_T_CHUNK = 8
_T_NBUF = 4
_T_AHEAD = 2


def _ring_copy(src_hbm, dst_hbm, buf, gsem, ssem, base, rows, chunk, nbuf,
               ahead):
    nchunks = rows // chunk

    def gather(j):
        g = pltpu.make_async_copy(
            src_hbm.at[pl.ds(base + j * chunk, chunk)], buf.at[j % nbuf], gsem)
        g.start()
        return g

    gaths, scats = [], []
    for j in range(min(ahead, nchunks)):
        gaths.append(gather(j))
    for i in range(nchunks):
        gaths[i].wait()
        s = pltpu.make_async_copy(
            buf.at[i % nbuf], dst_hbm.at[pl.ds(base + i * chunk, chunk)], ssem)
        s.start()
        scats.append(s)
        j = i + ahead
        if j < nchunks:
            if j >= nbuf:
                scats[j - nbuf].wait()
            gaths.append(gather(j))
    for s in scats[-min(nbuf, nchunks):]:
        s.wait()


def _scs_body(table_hbm, out_hbm, sbuf):
    rows_per_c = _SCS_ROWS // _NC
    cid = lax.axis_index("c")

    def inner(gsem, ssem):
        _ring_copy(table_hbm, out_hbm, sbuf, gsem, ssem, cid * rows_per_c,
                   rows_per_c, _S_CHUNK, _S_NBUF, _S_AHEAD)

    pl.run_scoped(inner, pltpu.SemaphoreType.DMA, pltpu.SemaphoreType.DMA)


def _tec_body(table_hbm, out_hbm, sbuf):
    del sbuf
    total_rows = table_hbm.shape[0]
    rows_per_w = (total_rows - _SCS_ROWS) // _NW
    wid = lax.axis_index("s") * _NC + lax.axis_index("c")
    d_model = table_hbm.shape[1]

    def inner(buf, gsem, ssem):
        _ring_copy(table_hbm, out_hbm, buf, gsem, ssem,
                   _SCS_ROWS + wid * rows_per_w, rows_per_w, _T_CHUNK, _T_NBUF,
                   _T_AHEAD)

    pl.run_scoped(inner, pltpu.VMEM((_T_NBUF, _T_CHUNK, d_model),
                                    table_hbm.dtype),
                  pltpu.SemaphoreType.DMA, pltpu.SemaphoreType.DMA)


def kernel(x, table):
    seq_len = x.shape[1]
    d_model = table.shape[1]
    scalar_mesh = plsc.ScalarSubcoreMesh(axis_name="c", num_cores=_NC)
    vector_mesh = plsc.VectorSubcoreMesh(core_axis_name="c",
                                         subcore_axis_name="s")
    out = mpmd.mpmd_map(
        [(scalar_mesh, _scs_body), (vector_mesh, _tec_body)],
        out_types=jax.ShapeDtypeStruct((seq_len, d_model), table.dtype),
        scratch_types=[
            pltpu.VMEM_SHARED((_S_NBUF, _S_CHUNK, d_model), table.dtype),
        ],
    )(table[:seq_len])
    return out[None]


# speedup vs baseline: 1.0096x; 1.0096x over previous
import functools

import jax
import jax.numpy as jnp
from jax import lax
from jax.experimental import pallas as pl
from jax.experimental.pallas import tpu as pltpu
from jax.experimental.pallas import tpu_sc as plsc
from jax._src.pallas import mpmd

_INFO = plsc.get_sparse_core_info()
_NC = _INFO.num_cores
_NS = _INFO.num_subcores
_NW = _NC * _NS

_SCS_ROWS = 1280

_S_CHUNK = 128
_S_NBUF = 3
_S_AHEAD = 2

_T_CHUNK = 8
_T_NBUF = 4
_T_AHEAD = 2


def _ring_copy(src_hbm, dst_hbm, buf, gsem, ssem, base, rows, chunk, nbuf,
               ahead):
    nchunks = rows // chunk

    def gather(j):
        g = pltpu.make_async_copy(
            src_hbm.at[pl.ds(base + j * chunk, chunk)], buf.at[j % nbuf], gsem)
        g.start()
        return g

    gaths, scats = [], []
    for j in range(min(ahead, nchunks)):
        gaths.append(gather(j))
    for i in range(nchunks):
        gaths[i].wait()
        s = pltpu.make_async_copy(
            buf.at[i % nbuf], dst_hbm.at[pl.ds(base + i * chunk, chunk)], ssem)
        s.start()
        scats.append(s)
        j = i + ahead
        if j < nchunks:
            if j >= nbuf:
                scats[j - nbuf].wait()
            gaths.append(gather(j))
    for s in scats[-min(nbuf, nchunks):]:
        s.wait()


def _scs_body(table_hbm, out_hbm, sbuf):
    rows_per_c = _SCS_ROWS // _NC
    cid = lax.axis_index("c")

    def inner(gsem, ssem):
        _ring_copy(table_hbm, out_hbm, sbuf, gsem, ssem, cid * rows_per_c,
                   rows_per_c, _S_CHUNK, _S_NBUF, _S_AHEAD)

    pl.run_scoped(inner, pltpu.SemaphoreType.DMA, pltpu.SemaphoreType.DMA)


def _tec_body(table_hbm, out_hbm, sbuf):
    del sbuf
    total_rows = table_hbm.shape[0]
    rows_per_w = (total_rows - _SCS_ROWS) // _NW
    wid = lax.axis_index("s") * _NC + lax.axis_index("c")
    d_model = table_hbm.shape[1]

    def inner(buf, gsem, ssem):
        _ring_copy(table_hbm, out_hbm, buf, gsem, ssem,
                   _SCS_ROWS + wid * rows_per_w, rows_per_w, _T_CHUNK, _T_NBUF,
                   _T_AHEAD)

    pl.run_scoped(inner, pltpu.VMEM((_T_NBUF, _T_CHUNK, d_model),
                                    table_hbm.dtype),
                  pltpu.SemaphoreType.DMA, pltpu.SemaphoreType.DMA)


def kernel(x, table):
    seq_len = x.shape[1]
    d_model = table.shape[1]
    scalar_mesh = plsc.ScalarSubcoreMesh(axis_name="c", num_cores=_NC)
    vector_mesh = plsc.VectorSubcoreMesh(core_axis_name="c",
                                         subcore_axis_name="s")
    out = mpmd.mpmd_map(
        [(scalar_mesh, _scs_body), (vector_mesh, _tec_body)],
        out_types=jax.ShapeDtypeStruct((seq_len, d_model), table.dtype),
        scratch_types=[
            pltpu.VMEM_SHARED((_S_NBUF, _S_CHUNK, d_model), table.dtype),
        ],
    )(table[:seq_len])
    return out[None]


# speedup vs baseline: 1.0126x; 1.0030x over previous
import functools

import jax
import jax.numpy as jnp
from jax import lax
from jax.experimental import pallas as pl
from jax.experimental.pallas import tpu as pltpu
from jax.experimental.pallas import tpu_sc as plsc
from jax._src.pallas import mpmd

_INFO = plsc.get_sparse_core_info()
_NC = _INFO.num_cores
_NS = _INFO.num_subcores
_NW = _NC * _NS

_SCS_ROWS = 1536

_S_CHUNK = 128
_S_NBUF = 3
_S_AHEAD = 2

_T_CHUNK = 8
_T_NBUF = 4
_T_AHEAD = 2


def _ring_copy(src_hbm, dst_hbm, buf, gsem, ssem, base, rows, chunk, nbuf,
               ahead):
    nchunks = rows // chunk

    def gather(j):
        g = pltpu.make_async_copy(
            src_hbm.at[pl.ds(base + j * chunk, chunk)], buf.at[j % nbuf], gsem)
        g.start()
        return g

    gaths, scats = [], []
    for j in range(min(ahead, nchunks)):
        gaths.append(gather(j))
    for i in range(nchunks):
        gaths[i].wait()
        s = pltpu.make_async_copy(
            buf.at[i % nbuf], dst_hbm.at[pl.ds(base + i * chunk, chunk)], ssem)
        s.start()
        scats.append(s)
        j = i + ahead
        if j < nchunks:
            if j >= nbuf:
                scats[j - nbuf].wait()
            gaths.append(gather(j))
    for s in scats[-min(nbuf, nchunks):]:
        s.wait()


def _scs_body(table_hbm, out_hbm, sbuf):
    rows_per_c = _SCS_ROWS // _NC
    cid = lax.axis_index("c")

    def inner(gsem, ssem):
        _ring_copy(table_hbm, out_hbm, sbuf, gsem, ssem, cid * rows_per_c,
                   rows_per_c, _S_CHUNK, _S_NBUF, _S_AHEAD)

    pl.run_scoped(inner, pltpu.SemaphoreType.DMA, pltpu.SemaphoreType.DMA)


def _tec_body(table_hbm, out_hbm, sbuf):
    del sbuf
    total_rows = table_hbm.shape[0]
    rows_per_w = (total_rows - _SCS_ROWS) // _NW
    wid = lax.axis_index("s") * _NC + lax.axis_index("c")
    d_model = table_hbm.shape[1]

    def inner(buf, gsem, ssem):
        _ring_copy(table_hbm, out_hbm, buf, gsem, ssem,
                   _SCS_ROWS + wid * rows_per_w, rows_per_w, _T_CHUNK, _T_NBUF,
                   _T_AHEAD)

    pl.run_scoped(inner, pltpu.VMEM((_T_NBUF, _T_CHUNK, d_model),
                                    table_hbm.dtype),
                  pltpu.SemaphoreType.DMA, pltpu.SemaphoreType.DMA)


def kernel(x, table):
    seq_len = x.shape[1]
    d_model = table.shape[1]
    scalar_mesh = plsc.ScalarSubcoreMesh(axis_name="c", num_cores=_NC)
    vector_mesh = plsc.VectorSubcoreMesh(core_axis_name="c",
                                         subcore_axis_name="s")
    out = mpmd.mpmd_map(
        [(scalar_mesh, _scs_body), (vector_mesh, _tec_body)],
        out_types=jax.ShapeDtypeStruct((seq_len, d_model), table.dtype),
        scratch_types=[
            pltpu.VMEM_SHARED((_S_NBUF, _S_CHUNK, d_model), table.dtype),
        ],
    )(table[:seq_len])
    return out[None]
